# baseline (speedup 1.0000x reference)
import functools

import jax
import jax.numpy as jnp
from jax import lax
from jax.experimental import pallas as pl
from jax.experimental.pallas import tpu as pltpu
from jax.experimental.pallas import tpu_sc as plsc

VOCAB = 1000000
VEC = 64
HID = 300
NCLS = 100
B = 1024
L = 200
TOK = B * L

NC = 2
NS = 16
NW = NC * NS
ROWS_W = TOK // NW
CHUNK = 800
NCH = ROWS_W // CHUNK

BB = 8
TB = BB * L


def _sc_gather(idx, table):
    mesh = plsc.VectorSubcoreMesh(core_axis_name="c", subcore_axis_name="s")

    @functools.partial(
        pl.kernel,
        mesh=mesh,
        compiler_params=pltpu.CompilerParams(use_tc_tiling_on_sc=False),
        out_type=jax.ShapeDtypeStruct((TOK, VEC), jnp.float32),
        scratch_types=[
            pltpu.VMEM((ROWS_W,), jnp.int32),
            pltpu.VMEM((CHUNK, VEC), jnp.float32),
            pltpu.VMEM((CHUNK, VEC), jnp.float32),
            pltpu.SemaphoreType.DMA,
            pltpu.SemaphoreType.DMA,
        ],
    )
    def gather_kernel(idx_hbm, table_hbm, out_hbm, idx_v, buf0, buf1, sem0, sem1):
        wid = lax.axis_index("s") * NC + lax.axis_index("c")
        base = wid * ROWS_W
        pltpu.sync_copy(idx_hbm.at[pl.ds(base, ROWS_W)], idx_v)
        bufs = (buf0, buf1)
        sems = (sem0, sem1)
        inflight = pltpu.async_copy(
            table_hbm.at[idx_v.at[pl.ds(0, CHUNK)]], bufs[0], sems[0])
        for c in range(NCH):
            nxt = None
            if c + 1 < NCH:
                nxt = pltpu.async_copy(
                    table_hbm.at[idx_v.at[pl.ds((c + 1) * CHUNK, CHUNK)]],
                    bufs[(c + 1) % 2], sems[(c + 1) % 2])
            inflight.wait()
            pltpu.sync_copy(bufs[c % 2],
                            out_hbm.at[pl.ds(base + c * CHUNK, CHUNK)])
            if nxt is not None:
                inflight = nxt

    return gather_kernel(idx, table)


def _tc_body(g_ref, w1_ref, b1_ref, wc_ref, bc_ref, lab_ref, out_ref):
    i = pl.program_id(0)
    g = g_ref[...]
    h = lax.dot_general(
        g.astype(jnp.bfloat16), w1_ref[...].astype(jnp.bfloat16),
        (((1,), (0,)), ((), ())), preferred_element_type=jnp.float32)
    h = jnp.maximum(h + b1_ref[...], 0.0)
    pre = lax.dot_general(
        h.astype(jnp.bfloat16), wc_ref[...].astype(jnp.bfloat16),
        (((1,), (0,)), ((), ())), preferred_element_type=jnp.float32)
    pre = pre + bc_ref[...]
    pre = jnp.max(pre.reshape(BB, L, NCLS), axis=1)

    m = jnp.max(pre, axis=-1, keepdims=True)
    z = jnp.sum(jnp.exp(pre - m), axis=-1, keepdims=True)
    log_z = m + jnp.log(z)
    onehot = lax.broadcasted_iota(jnp.int32, (BB, NCLS), 1) == lab_ref[...]
    ll = jnp.sum(jnp.where(onehot, pre, 0.0), axis=-1, keepdims=True)
    part = jnp.sum(log_z - ll) * (1.0 / B)

    @pl.when(i == 0)
    def _init():
        out_ref[...] = jnp.zeros((1, 1), jnp.float32)

    out_ref[...] += part


def _tc_loss(gathered, label, W1, b1, Wc, bc, interpret=False):
    out = pl.pallas_call(
        _tc_body,
        grid=(B // BB,),
        in_specs=[
            pl.BlockSpec((TB, VEC), lambda i: (i, 0)),
            pl.BlockSpec((VEC, HID), lambda i: (0, 0)),
            pl.BlockSpec((1, HID), lambda i: (0, 0)),
            pl.BlockSpec((HID, NCLS), lambda i: (0, 0)),
            pl.BlockSpec((1, NCLS), lambda i: (0, 0)),
            pl.BlockSpec((BB, 1), lambda i: (i, 0)),
        ],
        out_specs=pl.BlockSpec((1, 1), lambda i: (0, 0)),
        out_shape=jax.ShapeDtypeStruct((1, 1), jnp.float32),
        interpret=interpret,
    )(gathered, W1, b1.reshape(1, HID), Wc, bc.reshape(1, NCLS),
      label.reshape(B, 1).astype(jnp.int32))
    return out[0, 0]


def kernel(x, label, emb_table, W1, b1, Wc, bc):
    idx = x.reshape(TOK).astype(jnp.int32)
    gathered = _sc_gather(idx, emb_table)
    return _tc_loss(gathered, label, W1, b1, Wc, bc)
